# baseline (speedup 1.0000x reference)
import functools

import jax
import jax.numpy as jnp
from jax import lax
from jax.experimental import pallas as pl
from jax.experimental.pallas import tpu as pltpu
from jax.experimental.pallas import tpu_sc as plsc

T = 17
H = 512
TN = 34
TD = 512
BS = 512
BB = 128
G = BS // BB
RB = 1024
DH = 32
EPS = 1e-5


def _mm(a, b):
    return jnp.dot(a, b, preferred_element_type=jnp.float32)


def _ln_local(x, g, b):
    mu = jnp.mean(x, axis=-1, keepdims=True)
    var = jnp.mean((x - mu) ** 2, axis=-1, keepdims=True)
    return (x - mu) / jnp.sqrt(var + EPS) * g + b


def _full(shape):
    r = len(shape)
    return pl.BlockSpec(shape, lambda i, _r=r: (0,) * _r)


def _vq_body(E, cbT, cbn, idx_out):
    best = jnp.full((RB,), jnp.inf, jnp.float32)
    besti = jnp.zeros((RB,), jnp.int32)
    for c in range(4):
        ct = cbT[:, c * 512:(c + 1) * 512]
        cn = cbn[0, c * 512:(c + 1) * 512]
        d = cn[None, :] - 2.0 * _mm(E[...], ct)
        li = jnp.argmin(d, axis=1).astype(jnp.int32)
        lv = jnp.min(d, axis=1)
        upd = lv < best
        besti = jnp.where(upd, li + c * 512, besti)
        best = jnp.where(upd, lv, best)
    idx_out[...] = besti


def _vq_argmin(E, codebook):
    return pl.pallas_call(
        _vq_body,
        grid=(E.shape[0] // RB,),
        in_specs=[pl.BlockSpec((RB, TD), lambda i: (i, 0)),
                  _full((TD, 2048)), _full((1, 2048))],
        out_specs=pl.BlockSpec((RB,), lambda i: (i,)),
        out_shape=jax.ShapeDtypeStruct((E.shape[0],), jnp.int32),
        compiler_params=pltpu.CompilerParams(
            dimension_semantics=("arbitrary",)),
    )(E, codebook.T, jnp.sum(codebook ** 2, axis=1)[None, :])


def _sc_gather(codebook, idx):
    n = idx.shape[0]
    info = plsc.get_sparse_core_info()
    nw = info.num_cores * info.num_subcores
    per_w = n // nw
    chunks = [(0, 128), (128, 128), (256, 128), (384, 128), (512, 32)]
    mesh = plsc.VectorSubcoreMesh(core_axis_name="c", subcore_axis_name="s")

    @functools.partial(
        pl.kernel,
        out_type=jax.ShapeDtypeStruct((n, TD), jnp.float32),
        mesh=mesh,
        scratch_types=[
            pltpu.VMEM((128,), jnp.int32),
            pltpu.VMEM((128, TD), jnp.float32),
            pltpu.VMEM((32,), jnp.int32),
            pltpu.VMEM((32, TD), jnp.float32),
            pltpu.SemaphoreType.DMA,
        ],
    )
    def k(table_hbm, idx_hbm, out_hbm, idx_v, rows_v, idx_s, rows_s, sem):
        wid = lax.axis_index("s") * info.num_cores + lax.axis_index("c")
        base = wid * per_w
        for off, sz in chunks:
            iv, rv = (idx_v, rows_v) if sz == 128 else (idx_s, rows_s)
            pltpu.sync_copy(idx_hbm.at[pl.ds(base + off, sz)], iv)
            pltpu.async_copy(table_hbm.at[iv], rv, sem).wait()
            pltpu.sync_copy(rv, out_hbm.at[pl.ds(base + off, sz)])

    return k(codebook, idx)


def _dec_body(part, dtokWt, dtokb, dsW, dsb,
              l1g, l1b, l2g, l2b, tW1t, tb1, tW2t, tb2, cW1, cb1, cW2, cb2,
              dlng, dlnb, recW, recb, out):
    p = (_mm(dtokWt[...], part[...].reshape(TN, BB * TD)) + dtokb[...])
    p = p.reshape(T, BB, TD)
    x = (_mm(p.reshape(T * BB, TD), dsW[...]) + dsb[0][None, :])
    x = x.reshape(T, BB, DH)

    y = _ln_local(x, l1g[0][None, None, :], l1b[0][None, None, :])
    ym = y.reshape(T, BB * DH)
    t1 = jax.nn.gelu(_mm(tW1t[...], ym) + tb1[...])
    yt = (_mm(tW2t[...], t1) + tb2[...]).reshape(T, BB, DH)
    x1 = x + yt
    z = _ln_local(x1, l2g[0][None, None, :], l2b[0][None, None, :])
    zm = z.reshape(T * BB, DH)
    c1 = jax.nn.gelu(_mm(zm, cW1[...]) + cb1[0][None, :])
    c2 = (_mm(c1, cW2[...]) + cb2[0][None, :]).reshape(T, BB, DH)
    x = x1 + c2

    x = _ln_local(x, dlng[0][None, None, :], dlnb[0][None, None, :])
    r = (_mm(x.reshape(T * BB, DH), recW[...]) + recb[0][None, :])
    out[...] = r.reshape(T, BB, 2)


def _dec_call(partN, *ws):
    ins = [partN] + list(ws)
    specs = [pl.BlockSpec((TN, BB, TD), lambda i: (0, i, 0))]
    specs += [_full(a.shape) for a in ws]
    return pl.pallas_call(
        _dec_body,
        grid=(G,),
        in_specs=specs,
        out_specs=pl.BlockSpec((T, BB, 2), lambda i: (0, i, 0)),
        out_shape=jax.ShapeDtypeStruct((T, BS, 2), jnp.float32),
        compiler_params=pltpu.CompilerParams(
            dimension_semantics=("arbitrary",)),
    )(*ins)


def kernel(feature_map, joints, cls_logits, rand_mask, params, codebook):
    del cls_logits
    p = params

    ef = feature_map.mean(axis=1) @ p['start_W'] + p['start_b']
    vis = jnp.logical_and(rand_mask > 0.2, joints[:, :, -1] != 0)
    w = vis[..., None].astype(ef.dtype)
    ef = ef * w + p['invisible_token'] * (1.0 - w)
    for blk in p['enc']:
        y = _ln_local(ef, blk['ln1_g'], blk['ln1_b'])
        y = jnp.swapaxes(y, 1, 2)
        y = jax.nn.gelu(y @ blk['tW1'] + blk['tb1']) @ blk['tW2'] + blk['tb2']
        y = jnp.swapaxes(y, 1, 2)
        z = _ln_local(ef + y, blk['ln2_g'], blk['ln2_b'])
        z = jax.nn.gelu(z @ blk['cW1'] + blk['cb1']) @ blk['cW2'] + blk['cb2']
        ef = ef + y + z
    ef = _ln_local(ef, p['enc_ln_g'], p['enc_ln_b'])
    ef = jnp.swapaxes(ef, 1, 2)
    ef = ef @ p['tokW'] + p['tokb']
    ef = jnp.swapaxes(ef, 1, 2)
    E = (ef @ p['featW'] + p['featb']).reshape(BS * TN, TD)

    encoding_indices = _vq_argmin(E, codebook)
    idxN = encoding_indices.reshape(BS, TN).T.reshape(-1)
    partN = _sc_gather(codebook, idxN)

    dec = p['dec'][0]
    rec = _dec_call(
        partN.reshape(TN, BS, TD),
        p['dec_tokW'].T, p['dec_tokb'][:, None],
        p['dec_startW'], p['dec_startb'][None, :],
        dec['ln1_g'][None, :], dec['ln1_b'][None, :],
        dec['ln2_g'][None, :], dec['ln2_b'][None, :],
        dec['tW1'].T, dec['tb1'][:, None], dec['tW2'].T, dec['tb2'][:, None],
        dec['cW1'], dec['cb1'][None, :], dec['cW2'], dec['cb2'][None, :],
        p['dec_ln_g'][None, :], p['dec_ln_b'][None, :],
        p['recW'], p['recb'][None, :])

    recovered = jnp.transpose(rec, (1, 0, 2))
    return recovered, encoding_indices
